# baseline (speedup 1.0000x reference)
import jax
import jax.numpy as jnp
from jax.experimental import pallas as pl
from jax.experimental.pallas import tpu as pltpu

_N = 26
_NBUF = 8


def _body(idxt_ref, out_ref, scratch, sems):
    x = idxt_ref[...]
    for k in range(_N):
        buf = k % _NBUF
        if k >= _NBUF:
            pltpu.make_async_copy(
                scratch.at[buf], out_ref.at[k - _NBUF], sems.at[buf]
            ).wait()
        scratch[buf, :, :] = (x == k).astype(jnp.int32)
        pltpu.make_async_copy(scratch.at[buf], out_ref.at[k], sems.at[buf]).start()
    for k in range(max(_N - _NBUF, 0), _N):
        buf = k % _NBUF
        pltpu.make_async_copy(scratch.at[buf], out_ref.at[k], sems.at[buf]).wait()


def kernel(idxs_vec):
    b, l = idxs_vec.shape
    idxt = idxs_vec.T
    out3 = pl.pallas_call(
        _body,
        in_specs=[pl.BlockSpec((l, b), lambda: (0, 0))],
        out_specs=pl.BlockSpec(memory_space=pl.ANY),
        out_shape=jax.ShapeDtypeStruct((_N, l, b), jnp.int32),
        scratch_shapes=[
            pltpu.VMEM((_NBUF, l, b), jnp.int32),
            pltpu.SemaphoreType.DMA((_NBUF,)),
        ],
    )(idxt)
    return jnp.transpose(out3, (2, 1, 0))
